# baseline (speedup 1.0000x reference)
import functools

import jax
import jax.numpy as jnp
from jax import lax
from jax.experimental import pallas as pl
from jax.experimental.pallas import tpu as pltpu
from jax.experimental.pallas import tpu_sc as plsc

NUM_CORES = 2
NUM_SUBCORES = 16
LANES = 16
NW = NUM_CORES * NUM_SUBCORES

BATCH = 16384
EMB_DIM = 32
PACK = 128 // EMB_DIM
ROWS_PER_W = BATCH // NW
CHUNK = 128
NCHUNK = ROWS_PER_W // CHUNK
GROUPS = CHUNK // LANES

_mesh = plsc.VectorSubcoreMesh(core_axis_name="c", subcore_axis_name="s")


@functools.partial(
    pl.kernel,
    out_type=(
        jax.ShapeDtypeStruct((BATCH,), jnp.float32),
        jax.ShapeDtypeStruct((BATCH,), jnp.float32),
    ),
    mesh=_mesh,
    compiler_params=pltpu.CompilerParams(needs_layout_passes=False),
    scratch_types=[
        pltpu.VMEM((NCHUNK, CHUNK), jnp.int32),
        pltpu.VMEM((NCHUNK, CHUNK), jnp.int32),
        pltpu.VMEM((NCHUNK, CHUNK), jnp.int32),
        pltpu.VMEM((NCHUNK, CHUNK), jnp.int32),
        pltpu.VMEM((NCHUNK, CHUNK), jnp.int32),
        pltpu.VMEM((NCHUNK, CHUNK), jnp.int32),
        pltpu.VMEM((CHUNK, 128), jnp.float32),
        pltpu.VMEM((CHUNK, 128), jnp.float32),
        pltpu.VMEM((CHUNK, 128), jnp.float32),
        pltpu.VMEM((ROWS_PER_W,), jnp.float32),
        pltpu.VMEM((ROWS_PER_W,), jnp.float32),
        pltpu.SemaphoreType.DMA,
    ],
)
def _mfbpr_sc(users_hbm, pos_hbm, neg_hbm, ut_hbm, it_hbm,
              pos_out, neg_out,
              uidx, pidx, nidx, ubig, pbig, nbig,
              ubuf, pbuf, nbuf, psc, nsc, sem):
    wid = lax.axis_index("s") * NUM_CORES + lax.axis_index("c")
    blk = wid * NCHUNK

    pltpu.sync_copy(users_hbm.at[pl.ds(blk, NCHUNK)], uidx)
    pltpu.sync_copy(pos_hbm.at[pl.ds(blk, NCHUNK)], pidx)
    pltpu.sync_copy(neg_hbm.at[pl.ds(blk, NCHUNK)], nidx)

    def mkbig(i, c):
        for j in range(NCHUNK):
            sl = pl.ds(i * LANES, LANES)
            ubig[j, sl] = lax.shift_right_logical(uidx[j, sl], 2)
            pbig[j, sl] = lax.shift_right_logical(pidx[j, sl], 2)
            nbig[j, sl] = lax.shift_right_logical(nidx[j, sl], 2)
        return c

    lax.fori_loop(0, CHUNK // LANES, mkbig, 0)

    iota = lax.iota(jnp.int32, LANES)

    for j in range(NCHUNK):
        cps = [
            pltpu.async_copy(ut_hbm.at[ubig.at[j]], ubuf, sem),
            pltpu.async_copy(it_hbm.at[pbig.at[j]], pbuf, sem),
            pltpu.async_copy(it_hbm.at[nbig.at[j]], nbuf, sem),
        ]
        for cp in cps:
            cp.wait()

        def group(g, c):
            sl = pl.ds(g * LANES, LANES)
            ridx = g * LANES + iota
            offu = (uidx[j, sl] & (PACK - 1)) * EMB_DIM
            offp = (pidx[j, sl] & (PACK - 1)) * EMB_DIM
            offn = (nidx[j, sl] & (PACK - 1)) * EMB_DIM
            accp = jnp.zeros((LANES,), jnp.float32)
            accn = jnp.zeros((LANES,), jnp.float32)
            for d in range(EMB_DIM):
                u = plsc.load_gather(ubuf, [ridx, offu + d])
                p = plsc.load_gather(pbuf, [ridx, offp + d])
                n = plsc.load_gather(nbuf, [ridx, offn + d])
                accp = accp + u * p
                accn = accn + u * n
            psc[pl.ds(j * CHUNK + g * LANES, LANES)] = accp
            nsc[pl.ds(j * CHUNK + g * LANES, LANES)] = accn
            return c

        lax.fori_loop(0, GROUPS, group, 0)

    base = wid * ROWS_PER_W
    pltpu.sync_copy(psc, pos_out.at[pl.ds(base, ROWS_PER_W)])
    pltpu.sync_copy(nsc, neg_out.at[pl.ds(base, ROWS_PER_W)])


def kernel(users, pos_items, neg_items, user_table, item_table):
    u = users.astype(jnp.int32).reshape(NW * NCHUNK, CHUNK)
    p = pos_items.astype(jnp.int32).reshape(NW * NCHUNK, CHUNK)
    n = neg_items.astype(jnp.int32).reshape(NW * NCHUNK, CHUNK)
    ut = user_table.reshape(-1, PACK * EMB_DIM)
    it = item_table.reshape(-1, PACK * EMB_DIM)
    return _mfbpr_sc(u, p, n, ut, it)


# speedup vs baseline: 3.2053x; 3.2053x over previous
import functools

import jax
import jax.numpy as jnp
from jax import lax
from jax.experimental import pallas as pl
from jax.experimental.pallas import tpu as pltpu
from jax.experimental.pallas import tpu_sc as plsc

NUM_CORES = 2
NUM_SUBCORES = 16
LANES = 16
NW = NUM_CORES * NUM_SUBCORES

BATCH = 16384
EMB_DIM = 32
NROWS = 1000000
ROWS_PER_W = BATCH // NW
CHUNK = 128
NCHUNK = ROWS_PER_W // CHUNK
GROUPS = ROWS_PER_W // LANES

W = 65536
NBLK = 16
TRS = EMB_DIM // 8
FLAT = TRS * NBLK * 8 * W


def _dump_body(x_ref, o_ref):
    x = x_ref[...]
    o_ref[...] = x.reshape(8, W // 128, 128).swapaxes(0, 1).reshape(8 * W)


_dump = pl.pallas_call(
    _dump_body,
    grid=(TRS, NBLK),
    in_specs=[pl.BlockSpec((8, W), lambda t, c: (t, c))],
    out_specs=pl.BlockSpec((8 * W,), lambda t, c: (t * NBLK + c,)),
    out_shape=jax.ShapeDtypeStruct((FLAT,), jnp.float32),
)

_mesh = plsc.VectorSubcoreMesh(core_axis_name="c", subcore_axis_name="s")


@functools.partial(
    pl.kernel,
    out_type=(
        jax.ShapeDtypeStruct((BATCH,), jnp.float32),
        jax.ShapeDtypeStruct((BATCH,), jnp.float32),
    ),
    mesh=_mesh,
    compiler_params=pltpu.CompilerParams(needs_layout_passes=False),
    scratch_types=[
        pltpu.VMEM((NCHUNK, CHUNK), jnp.int32),
        pltpu.VMEM((NCHUNK, CHUNK), jnp.int32),
        pltpu.VMEM((NCHUNK, CHUNK), jnp.int32),
        pltpu.VMEM((EMB_DIM * NCHUNK, CHUNK), jnp.int32),
        pltpu.VMEM((EMB_DIM * NCHUNK, CHUNK), jnp.int32),
        pltpu.VMEM((EMB_DIM * NCHUNK, CHUNK), jnp.int32),
        pltpu.VMEM((EMB_DIM * ROWS_PER_W,), jnp.float32),
        pltpu.VMEM((EMB_DIM * ROWS_PER_W,), jnp.float32),
        pltpu.VMEM((EMB_DIM * ROWS_PER_W,), jnp.float32),
        pltpu.VMEM((ROWS_PER_W,), jnp.float32),
        pltpu.VMEM((ROWS_PER_W,), jnp.float32),
        pltpu.SemaphoreType.DMA,
    ],
)
def _mfbpr_sc(users_hbm, pos_hbm, neg_hbm, ut_hbm, it_hbm,
              pos_out, neg_out,
              uidx, pidx, nidx, ufidx, pfidx, nfidx,
              ubuf, pbuf, nbuf, psc, nsc, sem):
    wid = lax.axis_index("s") * NUM_CORES + lax.axis_index("c")
    blk = wid * NCHUNK

    pltpu.sync_copy(users_hbm.at[pl.ds(blk, NCHUNK)], uidx)
    pltpu.sync_copy(pos_hbm.at[pl.ds(blk, NCHUNK)], pidx)
    pltpu.sync_copy(neg_hbm.at[pl.ds(blk, NCHUNK)], nidx)

    def flat_base(u):
        return ((u >> 16) * (8 * W)
                + ((u >> 7) & 511) * 1024
                + (u & 127))

    def mkidx(i, c):
        for j in range(NCHUNK):
            sl = pl.ds(i * LANES, LANES)
            ub = flat_base(uidx[j, sl])
            pb = flat_base(pidx[j, sl])
            nb = flat_base(nidx[j, sl])

            def step(d, c2):
                doff = (d >> 3) * (NBLK * 8 * W) + (d & 7) * 128
                row = d * NCHUNK + j
                ufidx[row, sl] = ub + doff
                pfidx[row, sl] = pb + doff
                nfidx[row, sl] = nb + doff
                return c2

            lax.fori_loop(0, EMB_DIM, step, c)
        return c

    lax.fori_loop(0, CHUNK // LANES, mkidx, 0)

    copies = []
    for j in range(NCHUNK):
        for d in range(EMB_DIM):
            row = d * NCHUNK + j
            dst = pl.ds(d * ROWS_PER_W + j * CHUNK, CHUNK)
            copies.append(
                pltpu.async_copy(ut_hbm.at[ufidx.at[row]], ubuf.at[dst], sem))
            copies.append(
                pltpu.async_copy(it_hbm.at[pfidx.at[row]], pbuf.at[dst], sem))
            copies.append(
                pltpu.async_copy(it_hbm.at[nfidx.at[row]], nbuf.at[dst], sem))
    for cp in copies:
        cp.wait()

    def group(g, c):
        accp = jnp.zeros((LANES,), jnp.float32)
        accn = jnp.zeros((LANES,), jnp.float32)
        for d in range(EMB_DIM):
            sl = pl.ds(d * ROWS_PER_W + g * LANES, LANES)
            u = ubuf[sl]
            accp = accp + u * pbuf[sl]
            accn = accn + u * nbuf[sl]
        psc[pl.ds(g * LANES, LANES)] = accp
        nsc[pl.ds(g * LANES, LANES)] = accn
        return c

    lax.fori_loop(0, GROUPS, group, 0)

    base = wid * ROWS_PER_W
    pltpu.sync_copy(psc, pos_out.at[pl.ds(base, ROWS_PER_W)])
    pltpu.sync_copy(nsc, neg_out.at[pl.ds(base, ROWS_PER_W)])


def kernel(users, pos_items, neg_items, user_table, item_table):
    u = users.astype(jnp.int32).reshape(NW * NCHUNK, CHUNK)
    p = pos_items.astype(jnp.int32).reshape(NW * NCHUNK, CHUNK)
    n = neg_items.astype(jnp.int32).reshape(NW * NCHUNK, CHUNK)
    uflat = _dump(user_table.T)
    iflat = _dump(item_table.T)
    return _mfbpr_sc(u, p, n, uflat, iflat)
